# baseline (speedup 1.0000x reference)
import functools

import numpy as np
import jax
import jax.numpy as jnp
from jax.experimental import pallas as pl
from jax.experimental.pallas import tpu as pltpu


TN1 = 256
TN2 = 1024


def _split(a):
    hi = a.astype(jnp.bfloat16)
    lo = (a - hi.astype(jnp.float32)).astype(jnp.bfloat16)
    return hi, lo


def _dot3(ah, al, bh, bl):
    f = jnp.float32
    return (jnp.dot(ah, bh, preferred_element_type=f)
            + jnp.dot(ah, bl, preferred_element_type=f)
            + jnp.dot(al, bh, preferred_element_type=f))


def _k1_body(xyz1_ref, xyz2t_ref, p1_ref, p2h_ref, p2l_ref,
             w0ah_ref, w0al_ref, w0bh_ref, w0bl_ref, b0_ref,
             y0_ref, st_ref):
    t = pl.program_id(1)
    x = xyz1_ref[0]
    yz = xyz2t_ref[0]

    x0, x1, x2 = x[:, 0:1], x[:, 1:2], x[:, 2:3]
    z0, z1, z2 = yz[0:1, :], yz[1:2, :], yz[2:3, :]
    s1 = x0 * x0 + x1 * x1 + x2 * x2
    s2 = z0 * z0 + z1 * z1 + z2 * z2
    bf = lambda v: v.astype(jnp.bfloat16).astype(jnp.float32)
    dot = bf(x0) * bf(z0) + bf(x1) * bf(z1) + bf(x2) * bf(z2)
    dist = (-2.0 * dot + s1) + s2

    m0 = jnp.min(dist, axis=1, keepdims=True)
    d1 = jnp.where(dist == m0, jnp.inf, dist)
    m1 = jnp.min(d1, axis=1, keepdims=True)
    d2 = jnp.where(d1 == m1, jnp.inf, d1)
    m2 = jnp.min(d2, axis=1, keepdims=True)

    r0 = 1.0 / (m0 + 1e-8)
    r1 = 1.0 / (m1 + 1e-8)
    r2 = 1.0 / (m2 + 1e-8)
    inorm = 1.0 / (r0 + r1 + r2)
    w0 = r0 * inorm
    w1 = r1 * inorm
    w2 = r2 * inorm
    s_mat = jnp.where(dist == m0, w0,
                      jnp.where(dist == m1, w1,
                                jnp.where(dist == m2, w2, 0.0)))

    sh, sl = _split(s_mat)
    interp = _dot3(sh, sl, p2h_ref[0], p2l_ref[0])
    p1h, p1l = _split(p1_ref[0])
    ih, il = _split(interp)
    h = (_dot3(p1h, p1l, w0ah_ref[...], w0al_ref[...])
         + _dot3(ih, il, w0bh_ref[...], w0bl_ref[...])
         + b0_ref[...])
    y0_ref[0] = h

    @pl.when(t == 0)
    def _():
        st_ref[...] = jnp.zeros_like(st_ref)

    st_ref[0, 0:1, :] += jnp.sum(h, axis=0, keepdims=True)
    st_ref[0, 1:2, :] += jnp.sum(h * h, axis=0, keepdims=True)


def _k1i_body(xyz1_ref, xyz2t_ref, idx_ref, wgt_ref):
    b = pl.program_id(0)
    x = xyz1_ref[0]
    yz = xyz2t_ref[0]
    n2 = yz.shape[1]

    x0, x1, x2 = x[:, 0:1], x[:, 1:2], x[:, 2:3]
    z0, z1, z2 = yz[0:1, :], yz[1:2, :], yz[2:3, :]
    s1 = x0 * x0 + x1 * x1 + x2 * x2
    s2 = z0 * z0 + z1 * z1 + z2 * z2
    bf = lambda v: v.astype(jnp.bfloat16).astype(jnp.float32)
    dot = bf(x0) * bf(z0) + bf(x1) * bf(z1) + bf(x2) * bf(z2)
    dist = (-2.0 * dot + s1) + s2

    iota = jax.lax.broadcasted_iota(jnp.int32, dist.shape, 1)
    m0 = jnp.min(dist, axis=1, keepdims=True)
    d1 = jnp.where(dist == m0, jnp.inf, dist)
    m1 = jnp.min(d1, axis=1, keepdims=True)
    d2 = jnp.where(d1 == m1, jnp.inf, d1)
    m2 = jnp.min(d2, axis=1, keepdims=True)
    off = b * n2
    i0 = jnp.min(jnp.where(dist == m0, iota, n2), axis=1, keepdims=True) + off
    i1 = jnp.min(jnp.where(d1 == m1, iota, n2), axis=1, keepdims=True) + off
    i2 = jnp.min(jnp.where(d2 == m2, iota, n2), axis=1, keepdims=True) + off

    r0 = 1.0 / (m0 + 1e-8)
    r1 = 1.0 / (m1 + 1e-8)
    r2 = 1.0 / (m2 + 1e-8)
    inorm = 1.0 / (r0 + r1 + r2)
    idx_ref[0, :, 0:1] = i0
    idx_ref[0, :, 1:2] = i1
    idx_ref[0, :, 2:3] = i2
    wgt_ref[0, :, 0:1] = r0 * inorm
    wgt_ref[0, :, 1:2] = r1 * inorm
    wgt_ref[0, :, 2:3] = r2 * inorm


def _k1b_body(p1_ref, g_ref, wgt_ref, w0ah_ref, w0al_ref, w0bh_ref, w0bl_ref,
              b0_ref, y0_ref, st_ref):
    t = pl.program_id(1)
    w = wgt_ref[0]
    g = g_ref[0]
    interp = (w[:, 0:1] * g[:, 0, :]
              + w[:, 1:2] * g[:, 1, :]
              + w[:, 2:3] * g[:, 2, :])
    p1h, p1l = _split(p1_ref[0])
    ih, il = _split(interp)
    h = (_dot3(p1h, p1l, w0ah_ref[...], w0al_ref[...])
         + _dot3(ih, il, w0bh_ref[...], w0bl_ref[...])
         + b0_ref[...])
    y0_ref[0] = h

    @pl.when(t == 0)
    def _():
        st_ref[...] = jnp.zeros_like(st_ref)

    st_ref[0, 0:1, :] += jnp.sum(h, axis=0, keepdims=True)
    st_ref[0, 1:2, :] += jnp.sum(h * h, axis=0, keepdims=True)


def _sc_gather(table, idx):
    from jax.experimental.pallas import tpu_sc as plsc
    from jax import lax

    nc, ns = 2, 16
    nw = nc * ns
    bt = idx.shape[0]
    d = table.shape[1]
    chunk = 128
    b_per_w = bt // nw
    n_chunks = b_per_w // chunk
    mesh = plsc.VectorSubcoreMesh(core_axis_name="c", subcore_axis_name="s")

    @functools.partial(
        pl.kernel,
        out_type=jax.ShapeDtypeStruct((bt, d), jnp.float32),
        mesh=mesh,
        scratch_types=[
            pltpu.VMEM((chunk,), jnp.int32),
            pltpu.VMEM((chunk, d), jnp.float32),
            pltpu.SemaphoreType.DMA,
        ],
    )
    def k(table_hbm, idx_hbm, out_hbm, idx_v, rows_v, sem):
        wid = lax.axis_index("s") * nc + lax.axis_index("c")
        base = wid * b_per_w

        @pl.loop(0, n_chunks)
        def _(j):
            off = base + j * chunk
            pltpu.sync_copy(idx_hbm.at[pl.ds(off, chunk)], idx_v)
            pltpu.async_copy(table_hbm.at[idx_v], rows_v, sem).wait()
            pltpu.sync_copy(rows_v, out_hbm.at[pl.ds(off, chunk)])

    return k(table, idx)


def _k2_body(y0_ref, sc_ref, sh_ref, w1h_ref, w1l_ref, b1_ref, y1_ref, st_ref):
    t = pl.program_id(1)
    h = jnp.maximum(y0_ref[0] * sc_ref[...] + sh_ref[...], 0.0)
    hh, hl = _split(h)
    z = _dot3(hh, hl, w1h_ref[...], w1l_ref[...]) + b1_ref[...]
    y1_ref[0] = z

    @pl.when(t == 0)
    def _():
        st_ref[...] = jnp.zeros_like(st_ref)

    st_ref[0, 0:1, :] += jnp.sum(z, axis=0, keepdims=True)
    st_ref[0, 1:2, :] += jnp.sum(z * z, axis=0, keepdims=True)


def _k3_body(y1_ref, sc_ref, sh_ref, out_ref):
    out_ref[0] = jnp.maximum(y1_ref[0] * sc_ref[...] + sh_ref[...], 0.0)


def _bn_coeffs(st, n_total, gamma, beta):
    mean = st[0] / n_total
    var = st[1] / n_total - mean * mean
    scale = gamma / jnp.sqrt(var + 1e-5)
    shift = beta - mean * scale
    return scale.reshape(1, -1), shift.reshape(1, -1)


def _chain(xyz1, xyz2t, points1, points2, w0ah, w0al, w0bh, w0bl, b0r,
           w1h, w1l, b1r, gamma0, beta0, gamma1, beta1, n_total, axis_name):
    B, N1, _ = xyz1.shape
    N2 = xyz2t.shape[2]
    C1 = points1.shape[2]
    C2 = points2.shape[2]
    CO0 = w0ah.shape[1]
    CO1 = w1h.shape[1]

    tn1 = min(TN1, N1)
    tn2 = min(TN2, N1)
    nt1 = N1 // tn1
    idx3, wgt3 = pl.pallas_call(
        _k1i_body,
        grid=(B, nt1),
        in_specs=[
            pl.BlockSpec((1, tn1, 3), lambda b, t: (b, t, 0)),
            pl.BlockSpec((1, 3, N2), lambda b, t: (b, 0, 0)),
        ],
        out_specs=[
            pl.BlockSpec((1, tn1, 3), lambda b, t: (b, t, 0)),
            pl.BlockSpec((1, tn1, 3), lambda b, t: (b, t, 0)),
        ],
        out_shape=[
            jax.ShapeDtypeStruct((B, N1, 3), jnp.int32),
            jax.ShapeDtypeStruct((B, N1, 3), jnp.float32),
        ],
        compiler_params=pltpu.CompilerParams(dimension_semantics=("parallel", "arbitrary")),
    )(xyz1, xyz2t)

    gath = _sc_gather(points2.reshape(B * N2, C2), idx3.reshape(B * N1 * 3))
    gath = gath.reshape(B, N1, 3, C2)

    y0, st0 = pl.pallas_call(
        _k1b_body,
        grid=(B, nt1),
        in_specs=[
            pl.BlockSpec((1, tn1, C1), lambda b, t: (b, t, 0)),
            pl.BlockSpec((1, tn1, 3, C2), lambda b, t: (b, t, 0, 0)),
            pl.BlockSpec((1, tn1, 3), lambda b, t: (b, t, 0)),
            pl.BlockSpec((C1, CO0), lambda b, t: (0, 0)),
            pl.BlockSpec((C1, CO0), lambda b, t: (0, 0)),
            pl.BlockSpec((C2, CO0), lambda b, t: (0, 0)),
            pl.BlockSpec((C2, CO0), lambda b, t: (0, 0)),
            pl.BlockSpec((1, CO0), lambda b, t: (0, 0)),
        ],
        out_specs=[
            pl.BlockSpec((1, tn1, CO0), lambda b, t: (b, t, 0)),
            pl.BlockSpec((1, 8, CO0), lambda b, t: (b, 0, 0)),
        ],
        out_shape=[
            jax.ShapeDtypeStruct((B, N1, CO0), jnp.float32),
            jax.ShapeDtypeStruct((B, 8, CO0), jnp.float32),
        ],
        compiler_params=pltpu.CompilerParams(dimension_semantics=("parallel", "arbitrary")),
    )(points1, gath, wgt3, w0ah, w0al, w0bh, w0bl, b0r)

    st0s = jnp.sum(st0, axis=0)
    if axis_name is not None:
        st0s = jax.lax.psum(st0s, axis_name)
    sc0, sh0 = _bn_coeffs(st0s, n_total, gamma0, beta0)

    nt2 = N1 // tn2
    y1, st1 = pl.pallas_call(
        _k2_body,
        grid=(B, nt2),
        in_specs=[
            pl.BlockSpec((1, tn2, CO0), lambda b, t: (b, t, 0)),
            pl.BlockSpec((1, CO0), lambda b, t: (0, 0)),
            pl.BlockSpec((1, CO0), lambda b, t: (0, 0)),
            pl.BlockSpec((CO0, CO1), lambda b, t: (0, 0)),
            pl.BlockSpec((CO0, CO1), lambda b, t: (0, 0)),
            pl.BlockSpec((1, CO1), lambda b, t: (0, 0)),
        ],
        out_specs=[
            pl.BlockSpec((1, tn2, CO1), lambda b, t: (b, t, 0)),
            pl.BlockSpec((1, 8, CO1), lambda b, t: (b, 0, 0)),
        ],
        out_shape=[
            jax.ShapeDtypeStruct((B, N1, CO1), jnp.float32),
            jax.ShapeDtypeStruct((B, 8, CO1), jnp.float32),
        ],
        compiler_params=pltpu.CompilerParams(dimension_semantics=("parallel", "arbitrary")),
    )(y0, sc0, sh0, w1h, w1l, b1r)

    st1s = jnp.sum(st1, axis=0)
    if axis_name is not None:
        st1s = jax.lax.psum(st1s, axis_name)
    sc1, sh1 = _bn_coeffs(st1s, n_total, gamma1, beta1)

    out = pl.pallas_call(
        _k3_body,
        grid=(B, nt2),
        in_specs=[
            pl.BlockSpec((1, tn2, CO1), lambda b, t: (b, t, 0)),
            pl.BlockSpec((1, CO1), lambda b, t: (0, 0)),
            pl.BlockSpec((1, CO1), lambda b, t: (0, 0)),
        ],
        out_specs=pl.BlockSpec((1, tn2, CO1), lambda b, t: (b, t, 0)),
        out_shape=jax.ShapeDtypeStruct((B, N1, CO1), jnp.float32),
        compiler_params=pltpu.CompilerParams(dimension_semantics=("parallel", "parallel")),
    )(y1, sc1, sh1)

    return out


@jax.jit
def kernel(xyz1, xyz2, points1, points2, W0, b0, gamma0, beta0,
           W1, b1, gamma1, beta1):
    B, N1, _ = xyz1.shape
    C1 = points1.shape[2]
    CO0 = W0.shape[0]
    CO1 = W1.shape[0]
    n_total = B * N1

    xyz2t = jnp.swapaxes(xyz2, 1, 2)
    w0ah, w0al = _split(W0[:, :C1].T)
    w0bh, w0bl = _split(W0[:, C1:].T)
    w1h, w1l = _split(W1.T)
    b0r = b0.reshape(1, CO0)
    b1r = b1.reshape(1, CO1)

    return _chain(xyz1, xyz2t, points1, points2,
                  w0ah, w0al, w0bh, w0bl, b0r, w1h, w1l, b1r,
                  gamma0, beta0, gamma1, beta1, n_total, None)


# speedup vs baseline: 1.6632x; 1.6632x over previous
import functools

import numpy as np
import jax
import jax.numpy as jnp
from jax.experimental import pallas as pl
from jax.experimental.pallas import tpu as pltpu


TN1 = 256
TN2 = 1024


def _split(a):
    hi = a.astype(jnp.bfloat16)
    lo = (a - hi.astype(jnp.float32)).astype(jnp.bfloat16)
    return hi, lo


def _dot3(ah, al, bh, bl):
    f = jnp.float32
    return (jnp.dot(ah, bh, preferred_element_type=f)
            + jnp.dot(ah, bl, preferred_element_type=f)
            + jnp.dot(al, bh, preferred_element_type=f))


def _k1_body(xyz1_ref, xyz2t_ref, p1_ref, p2h_ref, p2l_ref,
             w0ah_ref, w0al_ref, w0bh_ref, w0bl_ref, b0_ref,
             y0_ref, st_ref):
    t = pl.program_id(1)
    x = xyz1_ref[0]
    yz = xyz2t_ref[0]

    x0, x1, x2 = x[:, 0:1], x[:, 1:2], x[:, 2:3]
    z0, z1, z2 = yz[0:1, :], yz[1:2, :], yz[2:3, :]
    s1 = x0 * x0 + x1 * x1 + x2 * x2
    s2 = z0 * z0 + z1 * z1 + z2 * z2
    dot = jnp.dot(x.astype(jnp.bfloat16), yz.astype(jnp.bfloat16),
                  preferred_element_type=jnp.float32)
    dist = (-2.0 * dot + s1) + s2

    m0 = jnp.min(dist, axis=1, keepdims=True)
    d1 = jnp.where(dist == m0, jnp.inf, dist)
    m1 = jnp.min(d1, axis=1, keepdims=True)
    d2 = jnp.where(d1 == m1, jnp.inf, d1)
    m2 = jnp.min(d2, axis=1, keepdims=True)

    r0 = 1.0 / (m0 + 1e-8)
    r1 = 1.0 / (m1 + 1e-8)
    r2 = 1.0 / (m2 + 1e-8)
    inorm = 1.0 / (r0 + r1 + r2)
    w0 = r0 * inorm
    w1 = r1 * inorm
    w2 = r2 * inorm
    s_mat = jnp.where(dist == m0, w0,
                      jnp.where(dist == m1, w1,
                                jnp.where(dist == m2, w2, 0.0)))

    sh, sl = _split(s_mat)
    interp = _dot3(sh, sl, p2h_ref[0], p2l_ref[0])
    p1h, p1l = _split(p1_ref[0])
    ih, il = _split(interp)
    h = (_dot3(p1h, p1l, w0ah_ref[...], w0al_ref[...])
         + _dot3(ih, il, w0bh_ref[...], w0bl_ref[...])
         + b0_ref[...])
    y0_ref[0] = h

    @pl.when(t == 0)
    def _():
        st_ref[...] = jnp.zeros_like(st_ref)

    st_ref[0, 0:1, :] += jnp.sum(h, axis=0, keepdims=True)
    st_ref[0, 1:2, :] += jnp.sum(h * h, axis=0, keepdims=True)


def _k2_body(y0_ref, sc_ref, sh_ref, w1h_ref, w1l_ref, b1_ref, y1_ref, st_ref):
    t = pl.program_id(1)
    h = jnp.maximum(y0_ref[0] * sc_ref[...] + sh_ref[...], 0.0)
    hh, hl = _split(h)
    z = _dot3(hh, hl, w1h_ref[...], w1l_ref[...]) + b1_ref[...]
    y1_ref[0] = z

    @pl.when(t == 0)
    def _():
        st_ref[...] = jnp.zeros_like(st_ref)

    st_ref[0, 0:1, :] += jnp.sum(z, axis=0, keepdims=True)
    st_ref[0, 1:2, :] += jnp.sum(z * z, axis=0, keepdims=True)


def _k3_body(y1_ref, sc_ref, sh_ref, out_ref):
    out_ref[0] = jnp.maximum(y1_ref[0] * sc_ref[...] + sh_ref[...], 0.0)


def _bn_coeffs(st, n_total, gamma, beta):
    mean = st[0] / n_total
    var = st[1] / n_total - mean * mean
    scale = gamma / jnp.sqrt(var + 1e-5)
    shift = beta - mean * scale
    return scale.reshape(1, -1), shift.reshape(1, -1)


def _chain(xyz1, xyz2t, points1, p2h, p2l, w0ah, w0al, w0bh, w0bl, b0r,
           w1h, w1l, b1r, gamma0, beta0, gamma1, beta1, n_total, axis_name):
    B, N1, _ = xyz1.shape
    N2 = xyz2t.shape[2]
    C1 = points1.shape[2]
    C2 = p2h.shape[2]
    CO0 = w0ah.shape[1]
    CO1 = w1h.shape[1]

    tn1 = min(TN1, N1)
    tn2 = min(TN2, N1)
    nt1 = N1 // tn1
    y0, st0 = pl.pallas_call(
        _k1_body,
        grid=(B, nt1),
        in_specs=[
            pl.BlockSpec((1, tn1, 3), lambda b, t: (b, t, 0)),
            pl.BlockSpec((1, 3, N2), lambda b, t: (b, 0, 0)),
            pl.BlockSpec((1, tn1, C1), lambda b, t: (b, t, 0)),
            pl.BlockSpec((1, N2, C2), lambda b, t: (b, 0, 0)),
            pl.BlockSpec((1, N2, C2), lambda b, t: (b, 0, 0)),
            pl.BlockSpec((C1, CO0), lambda b, t: (0, 0)),
            pl.BlockSpec((C1, CO0), lambda b, t: (0, 0)),
            pl.BlockSpec((C2, CO0), lambda b, t: (0, 0)),
            pl.BlockSpec((C2, CO0), lambda b, t: (0, 0)),
            pl.BlockSpec((1, CO0), lambda b, t: (0, 0)),
        ],
        out_specs=[
            pl.BlockSpec((1, tn1, CO0), lambda b, t: (b, t, 0)),
            pl.BlockSpec((1, 8, CO0), lambda b, t: (b, 0, 0)),
        ],
        out_shape=[
            jax.ShapeDtypeStruct((B, N1, CO0), jnp.float32),
            jax.ShapeDtypeStruct((B, 8, CO0), jnp.float32),
        ],
        compiler_params=pltpu.CompilerParams(dimension_semantics=("parallel", "arbitrary")),
    )(xyz1, xyz2t, points1, p2h, p2l, w0ah, w0al, w0bh, w0bl, b0r)

    st0s = jnp.sum(st0, axis=0)
    if axis_name is not None:
        st0s = jax.lax.psum(st0s, axis_name)
    sc0, sh0 = _bn_coeffs(st0s, n_total, gamma0, beta0)

    nt2 = N1 // tn2
    y1, st1 = pl.pallas_call(
        _k2_body,
        grid=(B, nt2),
        in_specs=[
            pl.BlockSpec((1, tn2, CO0), lambda b, t: (b, t, 0)),
            pl.BlockSpec((1, CO0), lambda b, t: (0, 0)),
            pl.BlockSpec((1, CO0), lambda b, t: (0, 0)),
            pl.BlockSpec((CO0, CO1), lambda b, t: (0, 0)),
            pl.BlockSpec((CO0, CO1), lambda b, t: (0, 0)),
            pl.BlockSpec((1, CO1), lambda b, t: (0, 0)),
        ],
        out_specs=[
            pl.BlockSpec((1, tn2, CO1), lambda b, t: (b, t, 0)),
            pl.BlockSpec((1, 8, CO1), lambda b, t: (b, 0, 0)),
        ],
        out_shape=[
            jax.ShapeDtypeStruct((B, N1, CO1), jnp.float32),
            jax.ShapeDtypeStruct((B, 8, CO1), jnp.float32),
        ],
        compiler_params=pltpu.CompilerParams(dimension_semantics=("parallel", "arbitrary")),
    )(y0, sc0, sh0, w1h, w1l, b1r)

    st1s = jnp.sum(st1, axis=0)
    if axis_name is not None:
        st1s = jax.lax.psum(st1s, axis_name)
    sc1, sh1 = _bn_coeffs(st1s, n_total, gamma1, beta1)

    out = pl.pallas_call(
        _k3_body,
        grid=(B, nt2),
        in_specs=[
            pl.BlockSpec((1, tn2, CO1), lambda b, t: (b, t, 0)),
            pl.BlockSpec((1, CO1), lambda b, t: (0, 0)),
            pl.BlockSpec((1, CO1), lambda b, t: (0, 0)),
        ],
        out_specs=pl.BlockSpec((1, tn2, CO1), lambda b, t: (b, t, 0)),
        out_shape=jax.ShapeDtypeStruct((B, N1, CO1), jnp.float32),
        compiler_params=pltpu.CompilerParams(dimension_semantics=("parallel", "parallel")),
    )(y1, sc1, sh1)

    return out


@jax.jit
def kernel(xyz1, xyz2, points1, points2, W0, b0, gamma0, beta0,
           W1, b1, gamma1, beta1):
    B, N1, _ = xyz1.shape
    C1 = points1.shape[2]
    CO0 = W0.shape[0]
    CO1 = W1.shape[0]
    n_total = B * N1

    xyz2t = jnp.swapaxes(xyz2, 1, 2)
    w0ah, w0al = _split(W0[:, :C1].T)
    w0bh, w0bl = _split(W0[:, C1:].T)
    w1h, w1l = _split(W1.T)
    p2h, p2l = _split(points2)
    b0r = b0.reshape(1, CO0)
    b1r = b1.reshape(1, CO1)

    return _chain(xyz1, xyz2t, points1, p2h, p2l,
                  w0ah, w0al, w0bh, w0bl, b0r, w1h, w1l, b1r,
                  gamma0, beta0, gamma1, beta1, n_total, None)


# speedup vs baseline: 1.8009x; 1.0828x over previous
import functools

import numpy as np
import jax
import jax.numpy as jnp
from jax.experimental import pallas as pl
from jax.experimental.pallas import tpu as pltpu


TN1 = 256
TN2 = 1024


def _split(a):
    hi = a.astype(jnp.bfloat16)
    lo = (a - hi.astype(jnp.float32)).astype(jnp.bfloat16)
    return hi, lo


def _dot3(ah, al, bh, bl):
    f = jnp.float32
    return (jnp.dot(ah, bh, preferred_element_type=f)
            + jnp.dot(ah, bl, preferred_element_type=f)
            + jnp.dot(al, bh, preferred_element_type=f))


def _k1_body(xyz1_ref, xyz2t_ref, p1_ref, p2h_ref, p2l_ref,
             w0ah_ref, w0al_ref, w0bh_ref, w0bl_ref, b0_ref,
             y0_ref, st_ref):
    t = pl.program_id(1)
    x = xyz1_ref[0]
    yz = xyz2t_ref[0]

    x0, x1, x2 = x[:, 0:1], x[:, 1:2], x[:, 2:3]
    z0, z1, z2 = yz[0:1, :], yz[1:2, :], yz[2:3, :]
    s1 = x0 * x0 + x1 * x1 + x2 * x2
    s2 = z0 * z0 + z1 * z1 + z2 * z2
    dot = jnp.dot(x.astype(jnp.bfloat16), yz.astype(jnp.bfloat16),
                  preferred_element_type=jnp.float32)
    dist = (-2.0 * dot + s1) + s2

    m0 = jnp.min(dist, axis=1, keepdims=True)
    d1 = jnp.where(dist == m0, jnp.inf, dist)
    m1 = jnp.min(d1, axis=1, keepdims=True)
    d2 = jnp.where(d1 == m1, jnp.inf, d1)
    m2 = jnp.min(d2, axis=1, keepdims=True)

    r0 = 1.0 / (m0 + 1e-8)
    r1 = 1.0 / (m1 + 1e-8)
    r2 = 1.0 / (m2 + 1e-8)
    inorm = 1.0 / (r0 + r1 + r2)
    w0 = r0 * inorm
    w1 = r1 * inorm
    w2 = r2 * inorm
    s_mat = jnp.where(dist == m0, w0,
                      jnp.where(dist == m1, w1,
                                jnp.where(dist == m2, w2, 0.0)))

    sh = s_mat.astype(jnp.bfloat16)
    interp = (jnp.dot(sh, p2h_ref[0], preferred_element_type=jnp.float32)
              + jnp.dot(sh, p2l_ref[0], preferred_element_type=jnp.float32))
    p1h, p1l = _split(p1_ref[0])
    ih, il = _split(interp)
    h = (_dot3(p1h, p1l, w0ah_ref[...], w0al_ref[...])
         + _dot3(ih, il, w0bh_ref[...], w0bl_ref[...])
         + b0_ref[...])
    y0_ref[0] = h

    @pl.when(t == 0)
    def _():
        st_ref[...] = jnp.zeros_like(st_ref)

    st_ref[0, 0:1, :] += jnp.sum(h, axis=0, keepdims=True)
    st_ref[0, 1:2, :] += jnp.sum(h * h, axis=0, keepdims=True)


def _k2_body(y0_ref, sc_ref, sh_ref, w1h_ref, w1l_ref, b1_ref, y1_ref, st_ref):
    t = pl.program_id(1)
    h = jnp.maximum(y0_ref[0] * sc_ref[...] + sh_ref[...], 0.0)
    hh, hl = _split(h)
    z = _dot3(hh, hl, w1h_ref[...], w1l_ref[...]) + b1_ref[...]
    y1_ref[0] = z

    @pl.when(t == 0)
    def _():
        st_ref[...] = jnp.zeros_like(st_ref)

    st_ref[0, 0:1, :] += jnp.sum(z, axis=0, keepdims=True)
    st_ref[0, 1:2, :] += jnp.sum(z * z, axis=0, keepdims=True)


def _k3_body(y1_ref, sc_ref, sh_ref, out_ref):
    out_ref[0] = jnp.maximum(y1_ref[0] * sc_ref[...] + sh_ref[...], 0.0)


def _bn_coeffs(st, n_total, gamma, beta):
    mean = st[0] / n_total
    var = st[1] / n_total - mean * mean
    scale = gamma / jnp.sqrt(var + 1e-5)
    shift = beta - mean * scale
    return scale.reshape(1, -1), shift.reshape(1, -1)


def _chain(xyz1, xyz2t, points1, p2h, p2l, w0ah, w0al, w0bh, w0bl, b0r,
           w1h, w1l, b1r, gamma0, beta0, gamma1, beta1, n_total, axis_name):
    B, N1, _ = xyz1.shape
    N2 = xyz2t.shape[2]
    C1 = points1.shape[2]
    C2 = p2h.shape[2]
    CO0 = w0ah.shape[1]
    CO1 = w1h.shape[1]

    tn1 = min(TN1, N1)
    tn2 = min(TN2, N1)
    nt1 = N1 // tn1
    y0, st0 = pl.pallas_call(
        _k1_body,
        grid=(B, nt1),
        in_specs=[
            pl.BlockSpec((1, tn1, 3), lambda b, t: (b, t, 0)),
            pl.BlockSpec((1, 3, N2), lambda b, t: (b, 0, 0)),
            pl.BlockSpec((1, tn1, C1), lambda b, t: (b, t, 0)),
            pl.BlockSpec((1, N2, C2), lambda b, t: (b, 0, 0)),
            pl.BlockSpec((1, N2, C2), lambda b, t: (b, 0, 0)),
            pl.BlockSpec((C1, CO0), lambda b, t: (0, 0)),
            pl.BlockSpec((C1, CO0), lambda b, t: (0, 0)),
            pl.BlockSpec((C2, CO0), lambda b, t: (0, 0)),
            pl.BlockSpec((C2, CO0), lambda b, t: (0, 0)),
            pl.BlockSpec((1, CO0), lambda b, t: (0, 0)),
        ],
        out_specs=[
            pl.BlockSpec((1, tn1, CO0), lambda b, t: (b, t, 0)),
            pl.BlockSpec((1, 8, CO0), lambda b, t: (b, 0, 0)),
        ],
        out_shape=[
            jax.ShapeDtypeStruct((B, N1, CO0), jnp.float32),
            jax.ShapeDtypeStruct((B, 8, CO0), jnp.float32),
        ],
        compiler_params=pltpu.CompilerParams(dimension_semantics=("parallel", "arbitrary")),
    )(xyz1, xyz2t, points1, p2h, p2l, w0ah, w0al, w0bh, w0bl, b0r)

    st0s = jnp.sum(st0, axis=0)
    if axis_name is not None:
        st0s = jax.lax.psum(st0s, axis_name)
    sc0, sh0 = _bn_coeffs(st0s, n_total, gamma0, beta0)

    nt2 = N1 // tn2
    y1, st1 = pl.pallas_call(
        _k2_body,
        grid=(B, nt2),
        in_specs=[
            pl.BlockSpec((1, tn2, CO0), lambda b, t: (b, t, 0)),
            pl.BlockSpec((1, CO0), lambda b, t: (0, 0)),
            pl.BlockSpec((1, CO0), lambda b, t: (0, 0)),
            pl.BlockSpec((CO0, CO1), lambda b, t: (0, 0)),
            pl.BlockSpec((CO0, CO1), lambda b, t: (0, 0)),
            pl.BlockSpec((1, CO1), lambda b, t: (0, 0)),
        ],
        out_specs=[
            pl.BlockSpec((1, tn2, CO1), lambda b, t: (b, t, 0)),
            pl.BlockSpec((1, 8, CO1), lambda b, t: (b, 0, 0)),
        ],
        out_shape=[
            jax.ShapeDtypeStruct((B, N1, CO1), jnp.float32),
            jax.ShapeDtypeStruct((B, 8, CO1), jnp.float32),
        ],
        compiler_params=pltpu.CompilerParams(dimension_semantics=("parallel", "arbitrary")),
    )(y0, sc0, sh0, w1h, w1l, b1r)

    st1s = jnp.sum(st1, axis=0)
    if axis_name is not None:
        st1s = jax.lax.psum(st1s, axis_name)
    sc1, sh1 = _bn_coeffs(st1s, n_total, gamma1, beta1)

    out = pl.pallas_call(
        _k3_body,
        grid=(B, nt2),
        in_specs=[
            pl.BlockSpec((1, tn2, CO1), lambda b, t: (b, t, 0)),
            pl.BlockSpec((1, CO1), lambda b, t: (0, 0)),
            pl.BlockSpec((1, CO1), lambda b, t: (0, 0)),
        ],
        out_specs=pl.BlockSpec((1, tn2, CO1), lambda b, t: (b, t, 0)),
        out_shape=jax.ShapeDtypeStruct((B, N1, CO1), jnp.float32),
        compiler_params=pltpu.CompilerParams(dimension_semantics=("parallel", "parallel")),
    )(y1, sc1, sh1)

    return out


@jax.jit
def kernel(xyz1, xyz2, points1, points2, W0, b0, gamma0, beta0,
           W1, b1, gamma1, beta1):
    B, N1, _ = xyz1.shape
    C1 = points1.shape[2]
    CO0 = W0.shape[0]
    CO1 = W1.shape[0]
    n_total = B * N1

    xyz2t = jnp.swapaxes(xyz2, 1, 2)
    w0ah, w0al = _split(W0[:, :C1].T)
    w0bh, w0bl = _split(W0[:, C1:].T)
    w1h, w1l = _split(W1.T)
    p2h, p2l = _split(points2)
    b0r = b0.reshape(1, CO0)
    b1r = b1.reshape(1, CO1)

    return _chain(xyz1, xyz2t, points1, p2h, p2l,
                  w0ah, w0al, w0bh, w0bl, b0r, w1h, w1l, b1r,
                  gamma0, beta0, gamma1, beta1, n_total, None)


# speedup vs baseline: 1.8620x; 1.0339x over previous
import functools

import numpy as np
import jax
import jax.numpy as jnp
from jax.experimental import pallas as pl
from jax.experimental.pallas import tpu as pltpu


TN1 = 256
TN2 = 1024


def _split(a):
    hi = a.astype(jnp.bfloat16)
    lo = (a - hi.astype(jnp.float32)).astype(jnp.bfloat16)
    return hi, lo


def _dot3(ah, al, bh, bl):
    f = jnp.float32
    return (jnp.dot(ah, bh, preferred_element_type=f)
            + jnp.dot(ah, bl, preferred_element_type=f)
            + jnp.dot(al, bh, preferred_element_type=f))


def _k1_body(xyz1_ref, xyz2t_ref, p1_ref, p2h_ref, p2l_ref,
             w0ah_ref, w0al_ref, w0bh_ref, w0bl_ref, b0_ref,
             y0_ref, st_ref):
    t = pl.program_id(1)
    x = xyz1_ref[0]
    yz = xyz2t_ref[0]

    x0, x1, x2 = x[:, 0:1], x[:, 1:2], x[:, 2:3]
    z0, z1, z2 = yz[0:1, :], yz[1:2, :], yz[2:3, :]
    s1 = x0 * x0 + x1 * x1 + x2 * x2
    s2 = z0 * z0 + z1 * z1 + z2 * z2
    dot = jnp.dot(x.astype(jnp.bfloat16), yz.astype(jnp.bfloat16),
                  preferred_element_type=jnp.float32)
    dist = (-2.0 * dot + s1) + s2

    m0 = jnp.min(dist, axis=1, keepdims=True)
    d1 = jnp.where(dist == m0, jnp.inf, dist)
    m1 = jnp.min(d1, axis=1, keepdims=True)
    d2 = jnp.where(d1 == m1, jnp.inf, d1)
    m2 = jnp.min(d2, axis=1, keepdims=True)

    r0 = 1.0 / (m0 + 1e-8)
    r1 = 1.0 / (m1 + 1e-8)
    r2 = 1.0 / (m2 + 1e-8)
    inorm = 1.0 / (r0 + r1 + r2)
    w0 = r0 * inorm
    w1 = r1 * inorm
    w2 = r2 * inorm
    s_mat = jnp.where(dist == m0, w0,
                      jnp.where(dist == m1, w1,
                                jnp.where(dist == m2, w2, 0.0)))

    sh = s_mat.astype(jnp.bfloat16)
    interp = (jnp.dot(sh, p2h_ref[0], preferred_element_type=jnp.float32)
              + jnp.dot(sh, p2l_ref[0], preferred_element_type=jnp.float32))
    h = (jnp.dot(p1_ref[0].astype(jnp.bfloat16), w0ah_ref[...],
                 preferred_element_type=jnp.float32)
         + jnp.dot(interp.astype(jnp.bfloat16), w0bh_ref[...],
                   preferred_element_type=jnp.float32)
         + b0_ref[...])
    y0_ref[0] = h

    @pl.when(t == 0)
    def _():
        st_ref[...] = jnp.zeros_like(st_ref)

    st_ref[0, 0:1, :] += jnp.sum(h, axis=0, keepdims=True)
    st_ref[0, 1:2, :] += jnp.sum(h * h, axis=0, keepdims=True)


def _k2_body(y0_ref, sc_ref, sh_ref, w1h_ref, w1l_ref, b1_ref, y1_ref, st_ref):
    t = pl.program_id(1)
    h = jnp.maximum(y0_ref[0] * sc_ref[...] + sh_ref[...], 0.0)
    z = jnp.dot(h.astype(jnp.bfloat16), w1h_ref[...],
                preferred_element_type=jnp.float32) + b1_ref[...]
    y1_ref[0] = z

    @pl.when(t == 0)
    def _():
        st_ref[...] = jnp.zeros_like(st_ref)

    st_ref[0, 0:1, :] += jnp.sum(z, axis=0, keepdims=True)
    st_ref[0, 1:2, :] += jnp.sum(z * z, axis=0, keepdims=True)


def _k3_body(y1_ref, sc_ref, sh_ref, out_ref):
    out_ref[0] = jnp.maximum(y1_ref[0] * sc_ref[...] + sh_ref[...], 0.0)


def _bn_coeffs(st, n_total, gamma, beta):
    mean = st[0] / n_total
    var = st[1] / n_total - mean * mean
    scale = gamma / jnp.sqrt(var + 1e-5)
    shift = beta - mean * scale
    return scale.reshape(1, -1), shift.reshape(1, -1)


def _chain(xyz1, xyz2t, points1, p2h, p2l, w0ah, w0al, w0bh, w0bl, b0r,
           w1h, w1l, b1r, gamma0, beta0, gamma1, beta1, n_total, axis_name):
    B, N1, _ = xyz1.shape
    N2 = xyz2t.shape[2]
    C1 = points1.shape[2]
    C2 = p2h.shape[2]
    CO0 = w0ah.shape[1]
    CO1 = w1h.shape[1]

    tn1 = min(TN1, N1)
    tn2 = min(TN2, N1)
    nt1 = N1 // tn1
    y0, st0 = pl.pallas_call(
        _k1_body,
        grid=(B, nt1),
        in_specs=[
            pl.BlockSpec((1, tn1, 3), lambda b, t: (b, t, 0)),
            pl.BlockSpec((1, 3, N2), lambda b, t: (b, 0, 0)),
            pl.BlockSpec((1, tn1, C1), lambda b, t: (b, t, 0)),
            pl.BlockSpec((1, N2, C2), lambda b, t: (b, 0, 0)),
            pl.BlockSpec((1, N2, C2), lambda b, t: (b, 0, 0)),
            pl.BlockSpec((C1, CO0), lambda b, t: (0, 0)),
            pl.BlockSpec((C1, CO0), lambda b, t: (0, 0)),
            pl.BlockSpec((C2, CO0), lambda b, t: (0, 0)),
            pl.BlockSpec((C2, CO0), lambda b, t: (0, 0)),
            pl.BlockSpec((1, CO0), lambda b, t: (0, 0)),
        ],
        out_specs=[
            pl.BlockSpec((1, tn1, CO0), lambda b, t: (b, t, 0)),
            pl.BlockSpec((1, 8, CO0), lambda b, t: (b, 0, 0)),
        ],
        out_shape=[
            jax.ShapeDtypeStruct((B, N1, CO0), jnp.float32),
            jax.ShapeDtypeStruct((B, 8, CO0), jnp.float32),
        ],
        compiler_params=pltpu.CompilerParams(dimension_semantics=("parallel", "arbitrary")),
    )(xyz1, xyz2t, points1, p2h, p2l, w0ah, w0al, w0bh, w0bl, b0r)

    st0s = jnp.sum(st0, axis=0)
    if axis_name is not None:
        st0s = jax.lax.psum(st0s, axis_name)
    sc0, sh0 = _bn_coeffs(st0s, n_total, gamma0, beta0)

    nt2 = N1 // tn2
    y1, st1 = pl.pallas_call(
        _k2_body,
        grid=(B, nt2),
        in_specs=[
            pl.BlockSpec((1, tn2, CO0), lambda b, t: (b, t, 0)),
            pl.BlockSpec((1, CO0), lambda b, t: (0, 0)),
            pl.BlockSpec((1, CO0), lambda b, t: (0, 0)),
            pl.BlockSpec((CO0, CO1), lambda b, t: (0, 0)),
            pl.BlockSpec((CO0, CO1), lambda b, t: (0, 0)),
            pl.BlockSpec((1, CO1), lambda b, t: (0, 0)),
        ],
        out_specs=[
            pl.BlockSpec((1, tn2, CO1), lambda b, t: (b, t, 0)),
            pl.BlockSpec((1, 8, CO1), lambda b, t: (b, 0, 0)),
        ],
        out_shape=[
            jax.ShapeDtypeStruct((B, N1, CO1), jnp.float32),
            jax.ShapeDtypeStruct((B, 8, CO1), jnp.float32),
        ],
        compiler_params=pltpu.CompilerParams(dimension_semantics=("parallel", "arbitrary")),
    )(y0, sc0, sh0, w1h, w1l, b1r)

    st1s = jnp.sum(st1, axis=0)
    if axis_name is not None:
        st1s = jax.lax.psum(st1s, axis_name)
    sc1, sh1 = _bn_coeffs(st1s, n_total, gamma1, beta1)

    out = pl.pallas_call(
        _k3_body,
        grid=(B, nt2),
        in_specs=[
            pl.BlockSpec((1, tn2, CO1), lambda b, t: (b, t, 0)),
            pl.BlockSpec((1, CO1), lambda b, t: (0, 0)),
            pl.BlockSpec((1, CO1), lambda b, t: (0, 0)),
        ],
        out_specs=pl.BlockSpec((1, tn2, CO1), lambda b, t: (b, t, 0)),
        out_shape=jax.ShapeDtypeStruct((B, N1, CO1), jnp.float32),
        compiler_params=pltpu.CompilerParams(dimension_semantics=("parallel", "parallel")),
    )(y1, sc1, sh1)

    return out


@jax.jit
def kernel(xyz1, xyz2, points1, points2, W0, b0, gamma0, beta0,
           W1, b1, gamma1, beta1):
    B, N1, _ = xyz1.shape
    C1 = points1.shape[2]
    CO0 = W0.shape[0]
    CO1 = W1.shape[0]
    n_total = B * N1

    xyz2t = jnp.swapaxes(xyz2, 1, 2)
    w0ah, w0al = _split(W0[:, :C1].T)
    w0bh, w0bl = _split(W0[:, C1:].T)
    w1h, w1l = _split(W1.T)
    p2h, p2l = _split(points2)
    b0r = b0.reshape(1, CO0)
    b1r = b1.reshape(1, CO1)

    return _chain(xyz1, xyz2t, points1, p2h, p2l,
                  w0ah, w0al, w0bh, w0bl, b0r, w1h, w1l, b1r,
                  gamma0, beta0, gamma1, beta1, n_total, None)


# speedup vs baseline: 1.8754x; 1.0072x over previous
import functools

import numpy as np
import jax
import jax.numpy as jnp
from jax.experimental import pallas as pl
from jax.experimental.pallas import tpu as pltpu


TN1 = 256
TN2 = 2048


def _split(a):
    hi = a.astype(jnp.bfloat16)
    lo = (a - hi.astype(jnp.float32)).astype(jnp.bfloat16)
    return hi, lo


def _dot3(ah, al, bh, bl):
    f = jnp.float32
    return (jnp.dot(ah, bh, preferred_element_type=f)
            + jnp.dot(ah, bl, preferred_element_type=f)
            + jnp.dot(al, bh, preferred_element_type=f))


def _k1_body(xyz1_ref, xyz2t_ref, p1_ref, p2h_ref, p2l_ref,
             w0ah_ref, w0al_ref, w0bh_ref, w0bl_ref, b0_ref,
             y0_ref, st_ref):
    t = pl.program_id(1)
    x = xyz1_ref[0]
    yz = xyz2t_ref[0]

    x0, x1, x2 = x[:, 0:1], x[:, 1:2], x[:, 2:3]
    z0, z1, z2 = yz[0:1, :], yz[1:2, :], yz[2:3, :]
    s1 = x0 * x0 + x1 * x1 + x2 * x2
    s2 = z0 * z0 + z1 * z1 + z2 * z2
    dot = jnp.dot(x.astype(jnp.bfloat16), yz.astype(jnp.bfloat16),
                  preferred_element_type=jnp.float32)
    dist = (-2.0 * dot + s1) + s2

    m0 = jnp.min(dist, axis=1, keepdims=True)
    d1 = jnp.where(dist == m0, jnp.inf, dist)
    m1 = jnp.min(d1, axis=1, keepdims=True)
    d2 = jnp.where(d1 == m1, jnp.inf, d1)
    m2 = jnp.min(d2, axis=1, keepdims=True)

    r0 = 1.0 / (m0 + 1e-8)
    r1 = 1.0 / (m1 + 1e-8)
    r2 = 1.0 / (m2 + 1e-8)
    inorm = 1.0 / (r0 + r1 + r2)
    w0 = r0 * inorm
    w1 = r1 * inorm
    w2 = r2 * inorm
    s_mat = jnp.where(dist == m0, w0,
                      jnp.where(dist == m1, w1,
                                jnp.where(dist == m2, w2, 0.0)))

    sh = s_mat.astype(jnp.bfloat16)
    interp = (jnp.dot(sh, p2h_ref[0], preferred_element_type=jnp.float32)
              + jnp.dot(sh, p2l_ref[0], preferred_element_type=jnp.float32))
    p1h, p1l = _split(p1_ref[0])
    ih, il = _split(interp)
    h = (_dot3(p1h, p1l, w0ah_ref[...], w0al_ref[...])
         + _dot3(ih, il, w0bh_ref[...], w0bl_ref[...])
         + b0_ref[...])
    y0_ref[0] = h

    @pl.when(t == 0)
    def _():
        st_ref[...] = jnp.zeros_like(st_ref)

    st_ref[0, 0:1, :] += jnp.sum(h, axis=0, keepdims=True)
    st_ref[0, 1:2, :] += jnp.sum(h * h, axis=0, keepdims=True)


def _k2_body(y0_ref, sc_ref, sh_ref, w1h_ref, w1l_ref, b1_ref, y1_ref, st_ref):
    t = pl.program_id(1)
    h = jnp.maximum(y0_ref[0] * sc_ref[...] + sh_ref[...], 0.0)
    hh, hl = _split(h)
    z = _dot3(hh, hl, w1h_ref[...], w1l_ref[...]) + b1_ref[...]
    y1_ref[0] = z

    @pl.when(t == 0)
    def _():
        st_ref[...] = jnp.zeros_like(st_ref)

    st_ref[0, 0:1, :] += jnp.sum(z, axis=0, keepdims=True)
    st_ref[0, 1:2, :] += jnp.sum(z * z, axis=0, keepdims=True)


def _k3_body(y1_ref, sc_ref, sh_ref, out_ref):
    out_ref[0] = jnp.maximum(y1_ref[0] * sc_ref[...] + sh_ref[...], 0.0)


def _bn_coeffs(st, n_total, gamma, beta):
    mean = st[0] / n_total
    var = st[1] / n_total - mean * mean
    scale = gamma / jnp.sqrt(var + 1e-5)
    shift = beta - mean * scale
    return scale.reshape(1, -1), shift.reshape(1, -1)


def _chain(xyz1, xyz2t, points1, p2h, p2l, w0ah, w0al, w0bh, w0bl, b0r,
           w1h, w1l, b1r, gamma0, beta0, gamma1, beta1, n_total, axis_name):
    B, N1, _ = xyz1.shape
    N2 = xyz2t.shape[2]
    C1 = points1.shape[2]
    C2 = p2h.shape[2]
    CO0 = w0ah.shape[1]
    CO1 = w1h.shape[1]

    tn1 = min(TN1, N1)
    tn2 = min(TN2, N1)
    nt1 = N1 // tn1
    y0, st0 = pl.pallas_call(
        _k1_body,
        grid=(B, nt1),
        in_specs=[
            pl.BlockSpec((1, tn1, 3), lambda b, t: (b, t, 0)),
            pl.BlockSpec((1, 3, N2), lambda b, t: (b, 0, 0)),
            pl.BlockSpec((1, tn1, C1), lambda b, t: (b, t, 0)),
            pl.BlockSpec((1, N2, C2), lambda b, t: (b, 0, 0)),
            pl.BlockSpec((1, N2, C2), lambda b, t: (b, 0, 0)),
            pl.BlockSpec((C1, CO0), lambda b, t: (0, 0)),
            pl.BlockSpec((C1, CO0), lambda b, t: (0, 0)),
            pl.BlockSpec((C2, CO0), lambda b, t: (0, 0)),
            pl.BlockSpec((C2, CO0), lambda b, t: (0, 0)),
            pl.BlockSpec((1, CO0), lambda b, t: (0, 0)),
        ],
        out_specs=[
            pl.BlockSpec((1, tn1, CO0), lambda b, t: (b, t, 0)),
            pl.BlockSpec((1, 8, CO0), lambda b, t: (b, 0, 0)),
        ],
        out_shape=[
            jax.ShapeDtypeStruct((B, N1, CO0), jnp.float32),
            jax.ShapeDtypeStruct((B, 8, CO0), jnp.float32),
        ],
        compiler_params=pltpu.CompilerParams(dimension_semantics=("parallel", "arbitrary")),
    )(xyz1, xyz2t, points1, p2h, p2l, w0ah, w0al, w0bh, w0bl, b0r)

    st0s = jnp.sum(st0, axis=0)
    if axis_name is not None:
        st0s = jax.lax.psum(st0s, axis_name)
    sc0, sh0 = _bn_coeffs(st0s, n_total, gamma0, beta0)

    nt2 = N1 // tn2
    y1, st1 = pl.pallas_call(
        _k2_body,
        grid=(B, nt2),
        in_specs=[
            pl.BlockSpec((1, tn2, CO0), lambda b, t: (b, t, 0)),
            pl.BlockSpec((1, CO0), lambda b, t: (0, 0)),
            pl.BlockSpec((1, CO0), lambda b, t: (0, 0)),
            pl.BlockSpec((CO0, CO1), lambda b, t: (0, 0)),
            pl.BlockSpec((CO0, CO1), lambda b, t: (0, 0)),
            pl.BlockSpec((1, CO1), lambda b, t: (0, 0)),
        ],
        out_specs=[
            pl.BlockSpec((1, tn2, CO1), lambda b, t: (b, t, 0)),
            pl.BlockSpec((1, 8, CO1), lambda b, t: (b, 0, 0)),
        ],
        out_shape=[
            jax.ShapeDtypeStruct((B, N1, CO1), jnp.float32),
            jax.ShapeDtypeStruct((B, 8, CO1), jnp.float32),
        ],
        compiler_params=pltpu.CompilerParams(dimension_semantics=("parallel", "arbitrary")),
    )(y0, sc0, sh0, w1h, w1l, b1r)

    st1s = jnp.sum(st1, axis=0)
    if axis_name is not None:
        st1s = jax.lax.psum(st1s, axis_name)
    sc1, sh1 = _bn_coeffs(st1s, n_total, gamma1, beta1)

    out = pl.pallas_call(
        _k3_body,
        grid=(B, nt2),
        in_specs=[
            pl.BlockSpec((1, tn2, CO1), lambda b, t: (b, t, 0)),
            pl.BlockSpec((1, CO1), lambda b, t: (0, 0)),
            pl.BlockSpec((1, CO1), lambda b, t: (0, 0)),
        ],
        out_specs=pl.BlockSpec((1, tn2, CO1), lambda b, t: (b, t, 0)),
        out_shape=jax.ShapeDtypeStruct((B, N1, CO1), jnp.float32),
        compiler_params=pltpu.CompilerParams(dimension_semantics=("parallel", "parallel")),
    )(y1, sc1, sh1)

    return out


@jax.jit
def kernel(xyz1, xyz2, points1, points2, W0, b0, gamma0, beta0,
           W1, b1, gamma1, beta1):
    B, N1, _ = xyz1.shape
    C1 = points1.shape[2]
    CO0 = W0.shape[0]
    CO1 = W1.shape[0]
    n_total = B * N1

    xyz2t = jnp.swapaxes(xyz2, 1, 2)
    w0ah, w0al = _split(W0[:, :C1].T)
    w0bh, w0bl = _split(W0[:, C1:].T)
    w1h, w1l = _split(W1.T)
    p2h, p2l = _split(points2)
    b0r = b0.reshape(1, CO0)
    b1r = b1.reshape(1, CO1)

    return _chain(xyz1, xyz2t, points1, p2h, p2l,
                  w0ah, w0al, w0bh, w0bl, b0r, w1h, w1l, b1r,
                  gamma0, beta0, gamma1, beta1, n_total, None)


# speedup vs baseline: 1.9119x; 1.0194x over previous
import functools

import numpy as np
import jax
import jax.numpy as jnp
from jax.experimental import pallas as pl
from jax.experimental.pallas import tpu as pltpu


TN1 = 256
TN2 = 4096


def _split(a):
    hi = a.astype(jnp.bfloat16)
    lo = (a - hi.astype(jnp.float32)).astype(jnp.bfloat16)
    return hi, lo


def _dot3(ah, al, bh, bl):
    f = jnp.float32
    return (jnp.dot(ah, bh, preferred_element_type=f)
            + jnp.dot(ah, bl, preferred_element_type=f)
            + jnp.dot(al, bh, preferred_element_type=f))


def _k1_body(xyz1_ref, xyz2t_ref, p1_ref, p2h_ref, p2l_ref,
             w0ah_ref, w0al_ref, w0bh_ref, w0bl_ref, b0_ref,
             y0_ref, st_ref):
    t = pl.program_id(1)
    x = xyz1_ref[0]
    yz = xyz2t_ref[0]

    x0, x1, x2 = x[:, 0:1], x[:, 1:2], x[:, 2:3]
    z0, z1, z2 = yz[0:1, :], yz[1:2, :], yz[2:3, :]
    s1 = x0 * x0 + x1 * x1 + x2 * x2
    s2 = z0 * z0 + z1 * z1 + z2 * z2
    dot = jnp.dot(x.astype(jnp.bfloat16), yz.astype(jnp.bfloat16),
                  preferred_element_type=jnp.float32)
    dist = (-2.0 * dot + s1) + s2

    m0 = jnp.min(dist, axis=1, keepdims=True)
    d1 = jnp.where(dist == m0, jnp.inf, dist)
    m1 = jnp.min(d1, axis=1, keepdims=True)
    d2 = jnp.where(d1 == m1, jnp.inf, d1)
    m2 = jnp.min(d2, axis=1, keepdims=True)

    r0 = 1.0 / (m0 + 1e-8)
    r1 = 1.0 / (m1 + 1e-8)
    r2 = 1.0 / (m2 + 1e-8)
    inorm = 1.0 / (r0 + r1 + r2)
    w0 = r0 * inorm
    w1 = r1 * inorm
    w2 = r2 * inorm
    s_mat = jnp.where(dist == m0, w0,
                      jnp.where(dist == m1, w1,
                                jnp.where(dist == m2, w2, 0.0)))

    sh = s_mat.astype(jnp.bfloat16)
    interp = (jnp.dot(sh, p2h_ref[0], preferred_element_type=jnp.float32)
              + jnp.dot(sh, p2l_ref[0], preferred_element_type=jnp.float32))
    p1h, p1l = _split(p1_ref[0])
    ih, il = _split(interp)
    h = (_dot3(p1h, p1l, w0ah_ref[...], w0al_ref[...])
         + _dot3(ih, il, w0bh_ref[...], w0bl_ref[...])
         + b0_ref[...])
    y0_ref[0] = h

    @pl.when(t == 0)
    def _():
        st_ref[...] = jnp.zeros_like(st_ref)

    st_ref[0, 0:1, :] += jnp.sum(h, axis=0, keepdims=True)
    st_ref[0, 1:2, :] += jnp.sum(h * h, axis=0, keepdims=True)


def _k2_body(y0_ref, sc_ref, sh_ref, w1h_ref, w1l_ref, b1_ref, y1_ref, st_ref):
    t = pl.program_id(1)
    h = jnp.maximum(y0_ref[0] * sc_ref[...] + sh_ref[...], 0.0)
    hh, hl = _split(h)
    z = _dot3(hh, hl, w1h_ref[...], w1l_ref[...]) + b1_ref[...]
    y1_ref[0] = z

    @pl.when(t == 0)
    def _():
        st_ref[...] = jnp.zeros_like(st_ref)

    st_ref[0, 0:1, :] += jnp.sum(z, axis=0, keepdims=True)
    st_ref[0, 1:2, :] += jnp.sum(z * z, axis=0, keepdims=True)


def _k3_body(y1_ref, sc_ref, sh_ref, out_ref):
    out_ref[0] = jnp.maximum(y1_ref[0] * sc_ref[...] + sh_ref[...], 0.0)


def _bn_coeffs(st, n_total, gamma, beta):
    mean = st[0] / n_total
    var = st[1] / n_total - mean * mean
    scale = gamma / jnp.sqrt(var + 1e-5)
    shift = beta - mean * scale
    return scale.reshape(1, -1), shift.reshape(1, -1)


def _chain(xyz1, xyz2t, points1, p2h, p2l, w0ah, w0al, w0bh, w0bl, b0r,
           w1h, w1l, b1r, gamma0, beta0, gamma1, beta1, n_total, axis_name):
    B, N1, _ = xyz1.shape
    N2 = xyz2t.shape[2]
    C1 = points1.shape[2]
    C2 = p2h.shape[2]
    CO0 = w0ah.shape[1]
    CO1 = w1h.shape[1]

    tn1 = min(TN1, N1)
    tn2 = min(TN2, N1)
    nt1 = N1 // tn1
    y0, st0 = pl.pallas_call(
        _k1_body,
        grid=(B, nt1),
        in_specs=[
            pl.BlockSpec((1, tn1, 3), lambda b, t: (b, t, 0)),
            pl.BlockSpec((1, 3, N2), lambda b, t: (b, 0, 0)),
            pl.BlockSpec((1, tn1, C1), lambda b, t: (b, t, 0)),
            pl.BlockSpec((1, N2, C2), lambda b, t: (b, 0, 0)),
            pl.BlockSpec((1, N2, C2), lambda b, t: (b, 0, 0)),
            pl.BlockSpec((C1, CO0), lambda b, t: (0, 0)),
            pl.BlockSpec((C1, CO0), lambda b, t: (0, 0)),
            pl.BlockSpec((C2, CO0), lambda b, t: (0, 0)),
            pl.BlockSpec((C2, CO0), lambda b, t: (0, 0)),
            pl.BlockSpec((1, CO0), lambda b, t: (0, 0)),
        ],
        out_specs=[
            pl.BlockSpec((1, tn1, CO0), lambda b, t: (b, t, 0)),
            pl.BlockSpec((1, 8, CO0), lambda b, t: (b, 0, 0)),
        ],
        out_shape=[
            jax.ShapeDtypeStruct((B, N1, CO0), jnp.float32),
            jax.ShapeDtypeStruct((B, 8, CO0), jnp.float32),
        ],
        compiler_params=pltpu.CompilerParams(dimension_semantics=("parallel", "arbitrary")),
    )(xyz1, xyz2t, points1, p2h, p2l, w0ah, w0al, w0bh, w0bl, b0r)

    st0s = jnp.sum(st0, axis=0)
    if axis_name is not None:
        st0s = jax.lax.psum(st0s, axis_name)
    sc0, sh0 = _bn_coeffs(st0s, n_total, gamma0, beta0)

    nt2 = N1 // tn2
    y1, st1 = pl.pallas_call(
        _k2_body,
        grid=(B, nt2),
        in_specs=[
            pl.BlockSpec((1, tn2, CO0), lambda b, t: (b, t, 0)),
            pl.BlockSpec((1, CO0), lambda b, t: (0, 0)),
            pl.BlockSpec((1, CO0), lambda b, t: (0, 0)),
            pl.BlockSpec((CO0, CO1), lambda b, t: (0, 0)),
            pl.BlockSpec((CO0, CO1), lambda b, t: (0, 0)),
            pl.BlockSpec((1, CO1), lambda b, t: (0, 0)),
        ],
        out_specs=[
            pl.BlockSpec((1, tn2, CO1), lambda b, t: (b, t, 0)),
            pl.BlockSpec((1, 8, CO1), lambda b, t: (b, 0, 0)),
        ],
        out_shape=[
            jax.ShapeDtypeStruct((B, N1, CO1), jnp.float32),
            jax.ShapeDtypeStruct((B, 8, CO1), jnp.float32),
        ],
        compiler_params=pltpu.CompilerParams(dimension_semantics=("parallel", "arbitrary")),
    )(y0, sc0, sh0, w1h, w1l, b1r)

    st1s = jnp.sum(st1, axis=0)
    if axis_name is not None:
        st1s = jax.lax.psum(st1s, axis_name)
    sc1, sh1 = _bn_coeffs(st1s, n_total, gamma1, beta1)

    out = pl.pallas_call(
        _k3_body,
        grid=(B, nt2),
        in_specs=[
            pl.BlockSpec((1, tn2, CO1), lambda b, t: (b, t, 0)),
            pl.BlockSpec((1, CO1), lambda b, t: (0, 0)),
            pl.BlockSpec((1, CO1), lambda b, t: (0, 0)),
        ],
        out_specs=pl.BlockSpec((1, tn2, CO1), lambda b, t: (b, t, 0)),
        out_shape=jax.ShapeDtypeStruct((B, N1, CO1), jnp.float32),
        compiler_params=pltpu.CompilerParams(dimension_semantics=("parallel", "parallel")),
    )(y1, sc1, sh1)

    return out


@jax.jit
def kernel(xyz1, xyz2, points1, points2, W0, b0, gamma0, beta0,
           W1, b1, gamma1, beta1):
    B, N1, _ = xyz1.shape
    C1 = points1.shape[2]
    CO0 = W0.shape[0]
    CO1 = W1.shape[0]
    n_total = B * N1

    xyz2t = jnp.swapaxes(xyz2, 1, 2)
    w0ah, w0al = _split(W0[:, :C1].T)
    w0bh, w0bl = _split(W0[:, C1:].T)
    w1h, w1l = _split(W1.T)
    p2h, p2l = _split(points2)
    b0r = b0.reshape(1, CO0)
    b1r = b1.reshape(1, CO1)

    return _chain(xyz1, xyz2t, points1, p2h, p2l,
                  w0ah, w0al, w0bh, w0bl, b0r, w1h, w1l, b1r,
                  gamma0, beta0, gamma1, beta1, n_total, None)


# speedup vs baseline: 1.9216x; 1.0051x over previous
import functools

import numpy as np
import jax
import jax.numpy as jnp
from jax.experimental import pallas as pl
from jax.experimental.pallas import tpu as pltpu


TN1 = 256
TN2 = 8192


def _split(a):
    hi = a.astype(jnp.bfloat16)
    lo = (a - hi.astype(jnp.float32)).astype(jnp.bfloat16)
    return hi, lo


def _dot3(ah, al, bh, bl):
    f = jnp.float32
    return (jnp.dot(ah, bh, preferred_element_type=f)
            + jnp.dot(ah, bl, preferred_element_type=f)
            + jnp.dot(al, bh, preferred_element_type=f))


def _k1_body(xyz1_ref, xyz2t_ref, p1_ref, p2h_ref, p2l_ref,
             w0ah_ref, w0al_ref, w0bh_ref, w0bl_ref, b0_ref,
             y0_ref, st_ref):
    t = pl.program_id(1)
    x = xyz1_ref[0]
    yz = xyz2t_ref[0]

    x0, x1, x2 = x[:, 0:1], x[:, 1:2], x[:, 2:3]
    z0, z1, z2 = yz[0:1, :], yz[1:2, :], yz[2:3, :]
    s1 = x0 * x0 + x1 * x1 + x2 * x2
    s2 = z0 * z0 + z1 * z1 + z2 * z2
    dot = jnp.dot(x.astype(jnp.bfloat16), yz.astype(jnp.bfloat16),
                  preferred_element_type=jnp.float32)
    dist = (-2.0 * dot + s1) + s2

    m0 = jnp.min(dist, axis=1, keepdims=True)
    d1 = jnp.where(dist == m0, jnp.inf, dist)
    m1 = jnp.min(d1, axis=1, keepdims=True)
    d2 = jnp.where(d1 == m1, jnp.inf, d1)
    m2 = jnp.min(d2, axis=1, keepdims=True)

    r0 = 1.0 / (m0 + 1e-8)
    r1 = 1.0 / (m1 + 1e-8)
    r2 = 1.0 / (m2 + 1e-8)
    inorm = 1.0 / (r0 + r1 + r2)
    w0 = r0 * inorm
    w1 = r1 * inorm
    w2 = r2 * inorm
    s_mat = jnp.where(dist == m0, w0,
                      jnp.where(dist == m1, w1,
                                jnp.where(dist == m2, w2, 0.0)))

    sh = s_mat.astype(jnp.bfloat16)
    interp = (jnp.dot(sh, p2h_ref[0], preferred_element_type=jnp.float32)
              + jnp.dot(sh, p2l_ref[0], preferred_element_type=jnp.float32))
    p1h, p1l = _split(p1_ref[0])
    ih, il = _split(interp)
    h = (_dot3(p1h, p1l, w0ah_ref[...], w0al_ref[...])
         + _dot3(ih, il, w0bh_ref[...], w0bl_ref[...])
         + b0_ref[...])
    y0_ref[0] = h

    @pl.when(t == 0)
    def _():
        st_ref[...] = jnp.zeros_like(st_ref)

    st_ref[0, 0:1, :] += jnp.sum(h, axis=0, keepdims=True)
    st_ref[0, 1:2, :] += jnp.sum(h * h, axis=0, keepdims=True)


def _k2_body(y0_ref, sc_ref, sh_ref, w1h_ref, w1l_ref, b1_ref, y1_ref, st_ref):
    t = pl.program_id(1)
    h = jnp.maximum(y0_ref[0] * sc_ref[...] + sh_ref[...], 0.0)
    hh, hl = _split(h)
    z = _dot3(hh, hl, w1h_ref[...], w1l_ref[...]) + b1_ref[...]
    y1_ref[0] = z

    @pl.when(t == 0)
    def _():
        st_ref[...] = jnp.zeros_like(st_ref)

    st_ref[0, 0:1, :] += jnp.sum(z, axis=0, keepdims=True)
    st_ref[0, 1:2, :] += jnp.sum(z * z, axis=0, keepdims=True)


def _k3_body(y1_ref, sc_ref, sh_ref, out_ref):
    out_ref[0] = jnp.maximum(y1_ref[0] * sc_ref[...] + sh_ref[...], 0.0)


def _bn_coeffs(st, n_total, gamma, beta):
    mean = st[0] / n_total
    var = st[1] / n_total - mean * mean
    scale = gamma / jnp.sqrt(var + 1e-5)
    shift = beta - mean * scale
    return scale.reshape(1, -1), shift.reshape(1, -1)


def _chain(xyz1, xyz2t, points1, p2h, p2l, w0ah, w0al, w0bh, w0bl, b0r,
           w1h, w1l, b1r, gamma0, beta0, gamma1, beta1, n_total, axis_name):
    B, N1, _ = xyz1.shape
    N2 = xyz2t.shape[2]
    C1 = points1.shape[2]
    C2 = p2h.shape[2]
    CO0 = w0ah.shape[1]
    CO1 = w1h.shape[1]

    tn1 = min(TN1, N1)
    tn2 = min(TN2, N1)
    nt1 = N1 // tn1
    y0, st0 = pl.pallas_call(
        _k1_body,
        grid=(B, nt1),
        in_specs=[
            pl.BlockSpec((1, tn1, 3), lambda b, t: (b, t, 0)),
            pl.BlockSpec((1, 3, N2), lambda b, t: (b, 0, 0)),
            pl.BlockSpec((1, tn1, C1), lambda b, t: (b, t, 0)),
            pl.BlockSpec((1, N2, C2), lambda b, t: (b, 0, 0)),
            pl.BlockSpec((1, N2, C2), lambda b, t: (b, 0, 0)),
            pl.BlockSpec((C1, CO0), lambda b, t: (0, 0)),
            pl.BlockSpec((C1, CO0), lambda b, t: (0, 0)),
            pl.BlockSpec((C2, CO0), lambda b, t: (0, 0)),
            pl.BlockSpec((C2, CO0), lambda b, t: (0, 0)),
            pl.BlockSpec((1, CO0), lambda b, t: (0, 0)),
        ],
        out_specs=[
            pl.BlockSpec((1, tn1, CO0), lambda b, t: (b, t, 0)),
            pl.BlockSpec((1, 8, CO0), lambda b, t: (b, 0, 0)),
        ],
        out_shape=[
            jax.ShapeDtypeStruct((B, N1, CO0), jnp.float32),
            jax.ShapeDtypeStruct((B, 8, CO0), jnp.float32),
        ],
        compiler_params=pltpu.CompilerParams(dimension_semantics=("parallel", "arbitrary")),
    )(xyz1, xyz2t, points1, p2h, p2l, w0ah, w0al, w0bh, w0bl, b0r)

    st0s = jnp.sum(st0, axis=0)
    if axis_name is not None:
        st0s = jax.lax.psum(st0s, axis_name)
    sc0, sh0 = _bn_coeffs(st0s, n_total, gamma0, beta0)

    nt2 = N1 // tn2
    y1, st1 = pl.pallas_call(
        _k2_body,
        grid=(B, nt2),
        in_specs=[
            pl.BlockSpec((1, tn2, CO0), lambda b, t: (b, t, 0)),
            pl.BlockSpec((1, CO0), lambda b, t: (0, 0)),
            pl.BlockSpec((1, CO0), lambda b, t: (0, 0)),
            pl.BlockSpec((CO0, CO1), lambda b, t: (0, 0)),
            pl.BlockSpec((CO0, CO1), lambda b, t: (0, 0)),
            pl.BlockSpec((1, CO1), lambda b, t: (0, 0)),
        ],
        out_specs=[
            pl.BlockSpec((1, tn2, CO1), lambda b, t: (b, t, 0)),
            pl.BlockSpec((1, 8, CO1), lambda b, t: (b, 0, 0)),
        ],
        out_shape=[
            jax.ShapeDtypeStruct((B, N1, CO1), jnp.float32),
            jax.ShapeDtypeStruct((B, 8, CO1), jnp.float32),
        ],
        compiler_params=pltpu.CompilerParams(dimension_semantics=("parallel", "arbitrary")),
    )(y0, sc0, sh0, w1h, w1l, b1r)

    st1s = jnp.sum(st1, axis=0)
    if axis_name is not None:
        st1s = jax.lax.psum(st1s, axis_name)
    sc1, sh1 = _bn_coeffs(st1s, n_total, gamma1, beta1)

    out = pl.pallas_call(
        _k3_body,
        grid=(B, nt2),
        in_specs=[
            pl.BlockSpec((1, tn2, CO1), lambda b, t: (b, t, 0)),
            pl.BlockSpec((1, CO1), lambda b, t: (0, 0)),
            pl.BlockSpec((1, CO1), lambda b, t: (0, 0)),
        ],
        out_specs=pl.BlockSpec((1, tn2, CO1), lambda b, t: (b, t, 0)),
        out_shape=jax.ShapeDtypeStruct((B, N1, CO1), jnp.float32),
        compiler_params=pltpu.CompilerParams(dimension_semantics=("parallel", "parallel")),
    )(y1, sc1, sh1)

    return out


@jax.jit
def kernel(xyz1, xyz2, points1, points2, W0, b0, gamma0, beta0,
           W1, b1, gamma1, beta1):
    B, N1, _ = xyz1.shape
    C1 = points1.shape[2]
    CO0 = W0.shape[0]
    CO1 = W1.shape[0]
    n_total = B * N1

    xyz2t = jnp.swapaxes(xyz2, 1, 2)
    w0ah, w0al = _split(W0[:, :C1].T)
    w0bh, w0bl = _split(W0[:, C1:].T)
    w1h, w1l = _split(W1.T)
    p2h, p2l = _split(points2)
    b0r = b0.reshape(1, CO0)
    b1r = b1.reshape(1, CO1)

    return _chain(xyz1, xyz2t, points1, p2h, p2l,
                  w0ah, w0al, w0bh, w0bl, b0r, w1h, w1l, b1r,
                  gamma0, beta0, gamma1, beta1, n_total, None)
